# baseline (speedup 1.0000x reference)
import functools

import numpy as np
import jax
import jax.numpy as jnp
from jax import lax
from jax.experimental import pallas as pl
from jax.experimental.pallas import tpu as pltpu
from jax.experimental.pallas import tpu_sc as plsc

B, C, H, W = 8, 64, 256, 256
NCLS = 8
L = H * W
HALF = 512
NPAIR = B * (NCLS - 1)
NSH = 3 * NPAIR
NJOB2 = 2 * NPAIR
SLEN = 1 << 22
JCAP = 1 << 21
SB = 16384
JB = 8192
NW = 32


def _mt_stream(n, seed=0):
    key = np.empty(624, dtype=np.uint64)
    s = seed & 0xFFFFFFFF
    for pos in range(624):
        key[pos] = s
        s = (1812433253 * (s ^ (s >> 30)) + pos + 1) & 0xFFFFFFFF
    key = key.astype(np.uint32)
    out = np.empty(n, dtype=np.uint32)
    got = 0
    up = np.uint32(0x80000000)
    lo = np.uint32(0x7FFFFFFF)
    ma = np.uint32(0x9908B0DF)
    while got < n:
        k2 = key.copy()
        y1 = (k2[0:227] & up) | (k2[1:228] & lo)
        k2[0:227] = k2[397:624] ^ (y1 >> 1) ^ np.where((y1 & 1) != 0, ma, np.uint32(0))
        y2 = (k2[227:454] & up) | (k2[228:455] & lo)
        k2[227:454] = k2[0:227] ^ (y2 >> 1) ^ np.where((y2 & 1) != 0, ma, np.uint32(0))
        y3 = (k2[454:623] & up) | (k2[455:624] & lo)
        k2[454:623] = k2[227:396] ^ (y3 >> 1) ^ np.where((y3 & 1) != 0, ma, np.uint32(0))
        y4 = (k2[623] & up) | (k2[0] & lo)
        k2[623] = k2[396] ^ (y4 >> 1) ^ (ma if (y4 & 1) != 0 else np.uint32(0))
        key = k2
        y = key.copy()
        y ^= y >> np.uint32(11)
        y ^= (y << np.uint32(7)) & np.uint32(0x9D2C5680)
        y ^= (y << np.uint32(15)) & np.uint32(0xEFC60000)
        y ^= y >> np.uint32(18)
        take = min(624, n - got)
        out[got:got + take] = y[:take]
        got += take
    return out


_STREAM = _mt_stream(SLEN).astype(np.int32)

_SC_PARAMS = pltpu.CompilerParams(
    needs_layout_passes=False, use_tc_tiling_on_sc=False)


def _iota():
    return lax.iota(jnp.int32, 16)


def _sc_mesh():
    return plsc.VectorSubcoreMesh(core_axis_name="c", subcore_axis_name="s")


def _k1_dilated_pred(preds):
    def body(p_ref, o_ref):
        p = p_ref[0]
        best = p[0]
        idx = jnp.zeros((H, W), jnp.int32)
        for c in range(1, NCLS):
            m = p[c] > best
            idx = jnp.where(m, c, idx)
            best = jnp.where(m, p[c], best)
        neg = jnp.full((1, W), -1, jnp.int32)
        up = jnp.concatenate([idx[1:], neg], axis=0)
        dn = jnp.concatenate([neg, idx[:-1]], axis=0)
        v = jnp.maximum(jnp.maximum(up, dn), idx)
        negc = jnp.full((H, 1), -1, jnp.int32)
        lf = jnp.concatenate([v[:, 1:], negc], axis=1)
        rt = jnp.concatenate([negc, v[:, :-1]], axis=1)
        o_ref[0] = jnp.maximum(jnp.maximum(lf, rt), v)

    return pl.pallas_call(
        body,
        grid=(B,),
        in_specs=[pl.BlockSpec((1, NCLS, H, W), lambda i: (i, 0, 0, 0))],
        out_specs=pl.BlockSpec((1, H, W), lambda i: (i, 0, 0)),
        out_shape=jax.ShapeDtypeStruct((B, H, W), jnp.int32),
    )(preds)


def _k2_compact(pdf, lbf):
    CH = 2048

    @functools.partial(
        pl.kernel, mesh=_sc_mesh(),
        out_type=[
            jax.ShapeDtypeStruct((NJOB2, L), jnp.int32),
            jax.ShapeDtypeStruct((NJOB2 * 16,), jnp.int32),
        ],
        scratch_types=[
            pltpu.VMEM((CH,), jnp.int32),
            pltpu.VMEM((CH,), jnp.int32),
            pltpu.VMEM((L + 16,), jnp.int32),
            pltpu.VMEM((16,), jnp.int32),
        ],
        compiler_params=_SC_PARAMS,
    )
    def k(pd_hbm, lb_hbm, sorted_hbm, cnt_hbm, pdv, lbv, outbuf, cv):
        wid = lax.axis_index("s") * 2 + lax.axis_index("c")
        iota = _iota()
        for jj in range(4):
            job = wid + NW * jj

            @pl.when(job < NJOB2)
            def _():
                pair = job >> 1
                kind = job & 1
                b = pair // (NCLS - 1)
                cl = pair - b * (NCLS - 1) + 1

                def blk(bb, cnt):
                    pltpu.sync_copy(pd_hbm.at[b, pl.ds(pl.multiple_of(bb * CH, CH), CH)], pdv)
                    pltpu.sync_copy(lb_hbm.at[b, pl.ds(pl.multiple_of(bb * CH, CH), CH)], lbv)

                    def inner(t, cnt):
                        lb16 = lbv[pl.ds(t * 16, 16)]
                        pd16 = pdv[pl.ds(t * 16, 16)]
                        pdeq = pd16 == cl
                        m = (lb16 == cl) & jnp.where(kind == 1, pdeq, ~pdeq)
                        csum = plsc.cumsum(m.astype(jnp.int32))
                        pos = jnp.where(m, cnt + csum - 1, L)
                        vals = _iota() + (bb * CH + t * 16)
                        plsc.store_scatter(outbuf, [pos], vals)
                        return cnt + csum[15]

                    return lax.fori_loop(0, CH // 16, inner, cnt)

                cnt = lax.fori_loop(0, L // CH, blk, jnp.int32(0))
                pltpu.sync_copy(outbuf.at[pl.ds(0, L)], sorted_hbm.at[job])
                cv[...] = jnp.where(iota == 0, cnt, 0)
                pltpu.sync_copy(cv, cnt_hbm.at[pl.ds(pl.multiple_of(job * 16, 16), 16)])

    return k(pdf, lbf)


def _k3_walk(stream, counts):
    @functools.partial(
        pl.kernel, mesh=_sc_mesh(),
        out_type=[
            jax.ShapeDtypeStruct((JCAP,), jnp.int32),
            jax.ShapeDtypeStruct((NSH, 16), jnp.int32),
        ],
        scratch_types=[
            pltpu.VMEM((SB,), jnp.int32),
            pltpu.VMEM((JB + 32,), jnp.int32),
            pltpu.VMEM((NJOB2 * 16,), jnp.int32),
            pltpu.VMEM((16,), jnp.int32),
        ],
        compiler_params=_SC_PARAMS,
    )
    def k(stream_hbm, cnt_hbm, jflat_hbm, offs_hbm, sbuf, jbuf, cntv, offv):
        wid = lax.axis_index("s") * 2 + lax.axis_index("c")

        @pl.when(wid == 0)
        def _():
            pltpu.sync_copy(cnt_hbm, cntv)
            pltpu.sync_copy(stream_hbm.at[pl.ds(0, SB)], sbuf)
            iota = _iota()

            def shuffle(s, carry):
                p, cb, lg, fb = carry
                pair = s // 3
                kk = s - 3 * pair
                nh = cntv[pl.ds(pl.multiple_of(32 * pair, 16), 16)][0]
                ne = cntv[pl.ds(pl.multiple_of(32 * pair + 16, 16), 16)][0]
                n = jnp.where(
                    kk == 0, nh,
                    jnp.where(kk == 1, ne,
                              jnp.maximum(nh - HALF, 0) + ne))
                offv[...] = jnp.where(iota == 0, fb + lg,
                                      jnp.where(iota == 1, n, 0))
                pltpu.sync_copy(offv, offs_hbm.at[s])

                def cond(c):
                    return c[0] >= 1

                def body(c):
                    i, p, cb, lg, fb = c
                    refill = p + 16 > cb + SB
                    ncb = jnp.where(refill, jnp.minimum(p & ~7, SLEN - SB), cb)

                    @pl.when(refill)
                    def _():
                        pltpu.sync_copy(stream_hbm.at[pl.ds(pl.multiple_of(ncb, 8), SB)], sbuf)

                    cb = ncb
                    y = sbuf[pl.ds(p - cb, 16)]
                    tl = i - _iota()
                    msk = tl | (tl >> 1)
                    msk = msk | (msk >> 2)
                    msk = msk | (msk >> 4)
                    msk = msk | (msk >> 8)
                    msk = msk | (msk >> 16)
                    v = y & msk
                    acc = (v <= tl) & (tl >= 1)
                    k16 = plsc.all_reduce_ffs(~acc)[0]
                    w = jbuf[pl.ds(lg, 16)]
                    jbuf[pl.ds(lg, 16)] = jnp.where(_iota() < k16, v, w)
                    vcnt = jnp.minimum(i, 16)
                    consumed = jnp.where(k16 >= vcnt, k16, k16 + 1)
                    p = p + consumed
                    i = i - k16
                    lg = lg + k16
                    flush = lg >= JB

                    @pl.when(flush)
                    def _():
                        pltpu.sync_copy(jbuf.at[pl.ds(0, JB)],
                                        jflat_hbm.at[pl.ds(pl.multiple_of(fb, JB), JB)])
                        jbuf[pl.ds(0, 16)] = jbuf[pl.ds(JB, 16)]
                        jbuf[pl.ds(16, 16)] = jbuf[pl.ds(JB + 16, 16)]

                    fb = jnp.where(flush, fb + JB, fb)
                    lg = jnp.where(flush, lg - JB, lg)
                    return (i, p, cb, lg, fb)

                _, p, cb, lg, fb = lax.while_loop(
                    cond, body, (n - 1, p, cb, lg, fb))
                lg = (lg + 7) & ~7
                flush = lg >= JB

                @pl.when(flush)
                def _():
                    pltpu.sync_copy(jbuf.at[pl.ds(0, JB)],
                                    jflat_hbm.at[pl.ds(pl.multiple_of(fb, JB), JB)])
                    jbuf[pl.ds(0, 16)] = jbuf[pl.ds(JB, 16)]
                    jbuf[pl.ds(16, 16)] = jbuf[pl.ds(JB + 16, 16)]

                fb = jnp.where(flush, fb + JB, fb)
                lg = jnp.where(flush, lg - JB, lg)
                return (p, cb, lg, fb)

            p, cb, lg, fb = lax.fori_loop(
                0, NSH, shuffle,
                (jnp.int32(0), jnp.int32(0), jnp.int32(0), jnp.int32(0)))
            pltpu.sync_copy(jbuf.at[pl.ds(0, JB)], jflat_hbm.at[pl.ds(pl.multiple_of(fb, JB), JB)])

    return k(stream, counts)


def _k4_apply(jflat, offs):
    @functools.partial(
        pl.kernel, mesh=_sc_mesh(),
        out_type=jax.ShapeDtypeStruct((NSH, L), jnp.int32),
        scratch_types=[
            pltpu.VMEM((L + 16,), jnp.int32),
            pltpu.VMEM((JB + 16,), jnp.int32),
            pltpu.VMEM((16,), jnp.int32),
        ],
        compiler_params=_SC_PARAMS,
    )
    def k(jflat_hbm, offs_hbm, perms_hbm, perm, jsb, offv):
        wid = lax.axis_index("s") * 2 + lax.axis_index("c")
        for jj in range(6):
            s = wid + NW * jj

            @pl.when(s < NSH)
            def _():
                pltpu.sync_copy(offs_hbm.at[s], offv)
                ov = offv[pl.ds(0, 16)]
                off = ov[0]
                n = ov[1]

                def init(t, _):
                    perm[pl.ds(t * 16, 16)] = _iota() + t * 16
                    return 0

                lax.fori_loop(0, L // 16, init, 0)

                def cond(c):
                    return c[0] < n - 1

                def body(c):
                    t = c[0]

                    @pl.when((t & (JB - 1)) == 0)
                    def _():
                        pltpu.sync_copy(jflat_hbm.at[pl.ds(pl.multiple_of(off + t, 8), JB)],
                                        jsb.at[pl.ds(0, JB)])

                    j = jsb[pl.ds(t & (JB - 1), 16)][0]
                    i = n - 1 - t
                    io = _iota()
                    gidx = jnp.where(io == 0, i, jnp.where(io == 1, j, L))
                    va = plsc.load_gather(perm, [gidx])
                    sidx = jnp.where(io == 0, j, jnp.where(io == 1, i, L))
                    plsc.store_scatter(perm, [sidx], va)
                    return (t + 1,)

                lax.while_loop(cond, body, (jnp.int32(0),))
                pltpu.sync_copy(perm.at[pl.ds(0, L)], perms_hbm.at[s])

    return k(jflat, offs)


def _k5_gather(perms, sorted_idx, counts, feat2d):
    perms2d = perms.reshape(-1, 16)
    sorted2d = sorted_idx.reshape(-1, 16)

    @functools.partial(
        pl.kernel, mesh=_sc_mesh(),
        out_type=jax.ShapeDtypeStruct((NPAIR, C, 2 * HALF), jnp.float32),
        scratch_types=[
            pltpu.VMEM((HALF,), jnp.int32),
            pltpu.VMEM((HALF,), jnp.int32),
            pltpu.VMEM((HALF,), jnp.int32),
            pltpu.VMEM((HALF,), jnp.int32),
            pltpu.VMEM((HALF,), jnp.int32),
            pltpu.VMEM((HALF,), jnp.int32),
            pltpu.VMEM((HALF,), jnp.int32),
            pltpu.VMEM((HALF, 16), jnp.int32),
            pltpu.VMEM((2 * HALF,), jnp.int32),
            pltpu.VMEM((2 * HALF,), jnp.int32),
            pltpu.VMEM((2 * HALF, 16), jnp.float32),
            pltpu.VMEM((2 * HALF,), jnp.float32),
            pltpu.VMEM((16,), jnp.int32),
            pltpu.SemaphoreType.DMA,
        ],
        compiler_params=_SC_PARAMS,
    )
    def k(cnt_hbm, perms2_hbm, sorted2_hbm, feat_hbm,
          fme_hbm, A, T, Bv, Cv, HBv, EBv, ridx, rows_i, idxall, fridx,
          rows_f, outrow, cv, sem):
        wid = lax.axis_index("s") * 2 + lax.axis_index("c")
        io = _iota()

        def gat512(tab, base, idx_buf, out_buf):
            def mk(t, _):
                x = idx_buf[pl.ds(t * 16, 16)]
                ridx[pl.ds(t * 16, 16)] = base + (x >> 4)
                return 0

            lax.fori_loop(0, HALF // 16, mk, 0)

            def dma(c2, _):
                pltpu.async_copy(tab.at[ridx.at[pl.ds(c2 * 128, 128)]],
                                 rows_i.at[pl.ds(c2 * 128, 128)], sem).wait()
                return 0

            lax.fori_loop(0, HALF // 128, dma, 0)

            def pick(t, _):
                x = idx_buf[pl.ds(t * 16, 16)] & 15
                out_buf[pl.ds(t * 16, 16)] = plsc.load_gather(
                    rows_i, [_iota() + t * 16, x])
                return 0

            lax.fori_loop(0, HALF // 16, pick, 0)

        def lin512(tab, rowbase, out_buf):
            pltpu.sync_copy(tab.at[pl.ds(pl.multiple_of(rowbase, 8), 32)],
                            rows_i.at[pl.ds(0, 32)])

            def pick(t, _):
                out_buf[pl.ds(t * 16, 16)] = plsc.load_gather(
                    rows_i, [_iota() * 0 + t, _iota()])
                return 0

            lax.fori_loop(0, HALF // 16, pick, 0)

        for jj in range(2):
            pair = wid + NW * jj

            @pl.when(pair < NPAIR)
            def _():
                b = pair // (NCLS - 1)
                s1 = 3 * pair
                p1b = s1 * (L // 16)
                p2b = (s1 + 1) * (L // 16)
                p3b = (s1 + 2) * (L // 16)
                hb = (2 * pair) * (L // 16)
                eb = (2 * pair + 1) * (L // 16)
                pltpu.sync_copy(cnt_hbm.at[pl.ds(pl.multiple_of(32 * pair, 16), 16)], cv)
                nh = cv[pl.ds(0, 16)][0]
                pltpu.sync_copy(cnt_hbm.at[pl.ds(pl.multiple_of(32 * pair + 16, 16), 16)], cv)
                ne = cv[pl.ds(0, 16)][0]
                hr_n = jnp.maximum(nh - HALF, 0)
                nr = hr_n + ne
                m_h = jnp.minimum(nh, HALF)
                m_r = jnp.minimum(nr, HALF)

                lin512(perms2_hbm, p1b, A)
                gat512(perms2_hbm, p1b, A, Bv)
                gat512(sorted2_hbm, hb, Bv, Cv)

                def wh(t, _):
                    idxall[pl.ds(t * 16, 16)] = jnp.clip(
                        Cv[pl.ds(t * 16, 16)], 0, L - 1)
                    return 0

                lax.fori_loop(0, HALF // 16, wh, 0)

                lin512(perms2_hbm, p3b, A)

                def t1(t, _):
                    T[pl.ds(t * 16, 16)] = jnp.clip(
                        A[pl.ds(t * 16, 16)] + HALF, 0, L - 1)
                    return 0

                lax.fori_loop(0, HALF // 16, t1, 0)
                gat512(perms2_hbm, p1b, T, Bv)
                gat512(perms2_hbm, p1b, Bv, Cv)
                gat512(sorted2_hbm, hb, Cv, HBv)

                def t2(t, _):
                    T[pl.ds(t * 16, 16)] = jnp.clip(
                        A[pl.ds(t * 16, 16)] - hr_n, 0, L - 1)
                    return 0

                lax.fori_loop(0, HALF // 16, t2, 0)
                gat512(perms2_hbm, p2b, T, Bv)
                gat512(sorted2_hbm, eb, Bv, EBv)

                def wr(t, _):
                    a = A[pl.ds(t * 16, 16)]
                    sel = jnp.where(a < hr_n, HBv[pl.ds(t * 16, 16)],
                                    EBv[pl.ds(t * 16, 16)])
                    idxall[pl.ds(HALF + t * 16, 16)] = jnp.clip(sel, 0, L - 1)
                    return 0

                lax.fori_loop(0, HALF // 16, wr, 0)

                def per_ch(ch, _):
                    cbase = (b * C + ch) * (L // 16)

                    def mk(t, _):
                        x = idxall[pl.ds(t * 16, 16)]
                        fridx[pl.ds(t * 16, 16)] = cbase + (x >> 4)
                        return 0

                    lax.fori_loop(0, 2 * HALF // 16, mk, 0)

                    def dma(c2, _):
                        pltpu.async_copy(
                            feat_hbm.at[fridx.at[pl.ds(c2 * 128, 128)]],
                            rows_f.at[pl.ds(c2 * 128, 128)], sem).wait()
                        return 0

                    lax.fori_loop(0, 2 * HALF // 128, dma, 0)

                    def pick(t, _):
                        x = idxall[pl.ds(t * 16, 16)] & 15
                        vals = plsc.load_gather(rows_f, [_iota() + t * 16, x])
                        col = _iota() + t * 16
                        lim = jnp.where(t < HALF // 16, m_h, m_r + HALF)
                        outrow[pl.ds(t * 16, 16)] = jnp.where(
                            col < lim, vals, 0.0)
                        return 0

                    lax.fori_loop(0, 2 * HALF // 16, pick, 0)
                    pltpu.sync_copy(outrow, fme_hbm.at[pair, ch])
                    return 0

                lax.fori_loop(0, C, per_ch, 0)

    return k(counts, perms2d, sorted2d, feat2d)


def kernel(feat_map, labels, preds):
    pd = _k1_dilated_pred(preds)
    pdf = pd.reshape(B, L)
    lbf = labels.reshape(B, L)
    sorted_idx, counts = _k2_compact(pdf, lbf)
    stream = jnp.asarray(_STREAM)
    jflat, offs = _k3_walk(stream, counts)
    perms = _k4_apply(jflat, offs)
    feat2d = feat_map.reshape(-1, 16)
    fme = _k5_gather(perms, sorted_idx, counts, feat2d)
    fl = jnp.tile(jnp.arange(1, NCLS, dtype=jnp.float32), B)
    return fme, fl
